# baseline (speedup 1.0000x reference)
import numpy as np
import jax
import jax.numpy as jnp
from jax import lax
from jax.experimental import pallas as pl
from jax.experimental.pallas import tpu as pltpu
from jax.experimental.pallas import tpu_sc as plsc

_P = 3
_D = 27
_DP = 32
_B = 4
_NQ = 256
_G = 1468
_GP = 1472
_NGRP = _GP // 16
_NW = 32
_QPW = (_B * _NQ) // _NW


def _cubic(t):
    a = -0.75
    at = jnp.abs(t)
    w1 = ((a + 2.0) * at - (a + 3.0)) * at * at + 1.0
    w2 = (((at - 5.0) * at + 8.0) * at - 4.0) * a
    return jnp.where(at <= 1.0, w1, jnp.where(at < 2.0, w2, 0.0))


def _resize1d(x, out_size, axis):
    in_size = x.shape[axis]
    o = jnp.arange(out_size, dtype=jnp.float32)
    src = o * ((in_size - 1) / (out_size - 1))
    i0 = jnp.floor(src).astype(jnp.int32)
    ts = src - i0.astype(jnp.float32)
    offs = jnp.arange(-1, 3)
    idx = jnp.clip(i0[:, None] + offs[None, :], 0, in_size - 1)
    w = _cubic(ts[:, None] - offs[None, :].astype(jnp.float32))
    xm = jnp.moveaxis(x, axis, -1)
    g = xm[..., idx]
    res = jnp.sum(g * w, axis=-1)
    return jnp.moveaxis(res, -1, axis)


def _resize(x, scale):
    _, _, h, w = x.shape
    x = _resize1d(x, int(round(h * scale)), 2)
    x = _resize1d(x, int(round(w * scale)), 3)
    return x


_NSRC = 3 * (48 * 48 + 24 * 24 + 12 * 12) + 2
_ZERO = _NSRC - 2
_BIG = _NSRC - 1


def _db_indices():
    sizes = [(0, 48), (3 * 2304, 24), (3 * 2304 + 3 * 576, 12)]
    rows = []

    def piece(base, s, i, j):
        n = (s - _P) // _P if (i or j) else s // _P
        for bi in range(n):
            for bj in range(n):
                row = []
                for c in range(3):
                    for pi in range(_P):
                        for pj in range(_P):
                            y = i + bi * _P + pi
                            x = j + bj * _P + pj
                            row.append(base + c * s * s + y * s + x)
                row.extend([_ZERO] * (_DP - _D))
                rows.append(row)

    for i in range(1, _P):
        for j in range(1, _P):
            for base, s in sizes:
                piece(base, s, i, j)
    for base, s in sizes:
        piece(base, s, 0, 0)
    for _ in range(_GP - _G):
        rows.append([_BIG] * _D + [_ZERO] * (_DP - _D))
    return np.asarray(rows, np.int32)


def _q_indices():
    rows = []
    for bi in range(16):
        for bj in range(16):
            row = []
            for c in range(3):
                for pi in range(_P):
                    for pj in range(_P):
                        row.append(c * 2304 + (bi * _P + pi) * 48 + bj * _P + pj)
            row.extend([3 * 2304] * (_DP - _D))
            rows.append(row)
    return np.asarray(rows, np.int32)


def _out_indices():
    idx = np.empty((3, 48, 48), np.int32)
    for c in range(3):
        for y in range(48):
            for x in range(48):
                q = (y // _P) * 16 + (x // _P)
                d = c * 9 + (y % _P) * _P + (x % _P)
                idx[c, y, x] = q * _DP + d
    return idx.reshape(-1)


_IDX_R = _db_indices()
_IDX_C = _IDX_R.T.copy()
_IDX_Q = _q_indices()
_IDX_OUT = _out_indices()


def _sc_nn_kernel(gc_hbm, gr_hbm, t_hbm, i_hbm,
                  sel_hbm, loss_hbm,
                  gc_v, gr_v, t_v, i_v, sel_v, loss_v):
    nc = 2
    wid = lax.axis_index("s") * nc + lax.axis_index("c")
    b = wid // (_NQ // _QPW)
    qbase = (wid % (_NQ // _QPW)) * _QPW

    pltpu.sync_copy(gc_hbm.at[b], gc_v)
    pltpu.sync_copy(gr_hbm.at[b], gr_v)
    pltpu.sync_copy(t_hbm.at[b, pl.ds(qbase, _QPW), :], t_v)
    pltpu.sync_copy(i_hbm.at[b, pl.ds(qbase, _QPW), :], i_v)

    lanes = lax.iota(jnp.int32, 16)

    def q_body(qi, lossacc):
        t_a = t_v[qi, pl.ds(0, 16)]
        t_b = t_v[qi, pl.ds(16, 16)]
        i_a = i_v[qi, pl.ds(0, 16)]
        i_b = i_v[qi, pl.ds(16, 16)]
        m_a = (t_a + i_a) * 0.5
        m_b = (t_b + i_b) * 0.5
        msp = []
        for d in range(_D):
            ms = m_a[d] if d < 16 else m_b[d - 16]
            msp.append(lax.broadcast(ms, (16,)))

        def g_body(gi, carry):
            mv, mg = carry
            accs = [jnp.zeros((16,), jnp.float32) for _ in range(4)]
            base = gi * 16
            for d in range(_D):
                gv = gc_v[d, pl.ds(base, 16)]
                df = msp[d] - gv
                accs[d % 4] = accs[d % 4] + df * df
            score = (accs[0] + accs[1]) + (accs[2] + accs[3])
            better = score < mv
            mv = jnp.where(better, score, mv)
            mg = jnp.where(better, lax.broadcast(gi, (16,)), mg)
            return mv, mg

        mv0 = jnp.full((16,), 3.0e38, jnp.float32)
        mg0 = jnp.zeros((16,), jnp.int32)
        mv, mg = lax.fori_loop(0, _NGRP, g_body, (mv0, mg0))

        gmin = jnp.min(mv)
        cand = mg * 16 + lanes
        masked = jnp.where(mv == gmin, cand, jnp.int32(2**30))
        cstar = jnp.min(masked)

        sel_a = gr_v[cstar, pl.ds(0, 16)]
        sel_b = gr_v[cstar, pl.ds(16, 16)]
        sel_v[qi, pl.ds(0, 16)] = sel_a
        sel_v[qi, pl.ds(16, 16)] = sel_b
        return lossacc + jnp.abs(i_a - sel_a) + jnp.abs(i_b - sel_b)

    lossacc = lax.fori_loop(0, _QPW, q_body, jnp.zeros((16,), jnp.float32))
    loss_v[...] = lossacc
    pltpu.sync_copy(sel_v, sel_hbm.at[b, pl.ds(qbase, _QPW), :])
    pltpu.sync_copy(loss_v, loss_hbm.at[wid])


def kernel(inp, tar):
    x2 = _resize(tar, 0.5)
    x4 = _resize(tar, 0.25)
    src = jnp.concatenate(
        [tar.reshape(_B, -1), x2.reshape(_B, -1), x4.reshape(_B, -1),
         jnp.zeros((_B, 1), jnp.float32),
         jnp.full((_B, 1), 1e9, jnp.float32)], axis=1)
    gr = jnp.take(src, _IDX_R, axis=1)
    gc = jnp.take(src, _IDX_C, axis=1)
    tq = jnp.concatenate(
        [tar.reshape(_B, -1), jnp.zeros((_B, 1), jnp.float32)], axis=1)
    iq = jnp.concatenate(
        [inp.reshape(_B, -1), jnp.zeros((_B, 1), jnp.float32)], axis=1)
    tr = jnp.take(tq, _IDX_Q, axis=1)
    ir = jnp.take(iq, _IDX_Q, axis=1)

    mesh = plsc.VectorSubcoreMesh(core_axis_name="c", subcore_axis_name="s")
    sel, lossp = pl.kernel(
        _sc_nn_kernel,
        mesh=mesh,
        compiler_params=pltpu.CompilerParams(
            needs_layout_passes=False, use_tc_tiling_on_sc=False),
        out_type=[
            jax.ShapeDtypeStruct((_B, _NQ, _DP), jnp.float32),
            jax.ShapeDtypeStruct((_NW, 16), jnp.float32),
        ],
        scratch_types=[
            pltpu.VMEM((_DP, _GP), jnp.float32),
            pltpu.VMEM((_GP, _DP), jnp.float32),
            pltpu.VMEM((_QPW, _DP), jnp.float32),
            pltpu.VMEM((_QPW, _DP), jnp.float32),
            pltpu.VMEM((_QPW, _DP), jnp.float32),
            pltpu.VMEM((16,), jnp.float32),
        ],
    )(gc, gr, tr, ir)

    sel_img = jnp.take(sel.reshape(_B, -1), _IDX_OUT, axis=1)
    sel_img = sel_img.reshape(_B, 3, 48, 48)
    loss = lossp.sum() / (_B * _NQ * _D)
    return loss, sel_img


# speedup vs baseline: 3.9245x; 3.9245x over previous
import numpy as np
import jax
import jax.numpy as jnp
from jax import lax
from jax.experimental import pallas as pl
from jax.experimental.pallas import tpu as pltpu
from jax.experimental.pallas import tpu_sc as plsc

_P = 3
_D = 27
_DP = 32
_B = 4
_NQ = 256
_G = 1468
_GP = 1472
_NGRP = _GP // 16
_NW = 32
_QPW = (_B * _NQ) // _NW

_B2 = 3 * 48 * 48
_B4 = _B2 + 3 * 24 * 24
_NSRC0 = _B4 + 3 * 12 * 12
_NSRC = _NSRC0 + 320


def _cubic(t):
    a = -0.75
    at = jnp.abs(t)
    w1 = ((a + 2.0) * at - (a + 3.0)) * at * at + 1.0
    w2 = (((at - 5.0) * at + 8.0) * at - 4.0) * a
    return jnp.where(at <= 1.0, w1, jnp.where(at < 2.0, w2, 0.0))


def _resize1d(x, out_size, axis):
    in_size = x.shape[axis]
    o = jnp.arange(out_size, dtype=jnp.float32)
    src = o * ((in_size - 1) / (out_size - 1))
    i0 = jnp.floor(src).astype(jnp.int32)
    ts = src - i0.astype(jnp.float32)
    offs = jnp.arange(-1, 3)
    idx = jnp.clip(i0[:, None] + offs[None, :], 0, in_size - 1)
    w = _cubic(ts[:, None] - offs[None, :].astype(jnp.float32))
    xm = jnp.moveaxis(x, axis, -1)
    g = xm[..., idx]
    res = jnp.sum(g * w, axis=-1)
    return jnp.moveaxis(res, -1, axis)


def _resize(x, scale):
    _, _, h, w = x.shape
    x = _resize1d(x, int(round(h * scale)), 2)
    x = _resize1d(x, int(round(w * scale)), 3)
    return x


def _reassemble(t, h, w, p=_P):
    b, _, c, ph, pw = t.shape
    nh, nw = h // p, w // p
    t = t.reshape(b, nh, nw, c, ph, pw)
    t = jnp.transpose(t, (0, 3, 1, 4, 2, 5))
    return t.reshape(b, c, h, w)


def _base_addrs():
    sizes = [(0, 48), (_B2, 24), (_B4, 12)]
    rows = []

    def piece(base, s, i, j):
        n = (s - _P) // _P if (i or j) else s // _P
        for bi in range(n):
            for bj in range(n):
                rows.append(base + (i + bi * _P) * s + j + bj * _P)

    for i in range(1, _P):
        for j in range(1, _P):
            for base, s in sizes:
                piece(base, s, i, j)
    for base, s in sizes:
        piece(base, s, 0, 0)
    assert len(rows) == _G
    rows.extend([_NSRC0] * (_GP - _G))
    return np.asarray(rows, np.int32)


def _dim_offsets():
    offs = []
    for s in (48, 24, 12):
        offs.append([c * s * s + pi * s + pj
                     for c in range(3) for pi in range(_P) for pj in range(_P)])
    flat = np.asarray(offs, np.int32).reshape(-1)
    flat = np.concatenate([flat, np.zeros(96 - flat.size, np.int32)])
    sel = np.empty((6, 16), np.int32)
    for s in range(3):
        sel[2 * s] = offs[s][:16]
        sel[2 * s + 1] = offs[s][16:] + [offs[s][26]] * (_DP - _D)
    return flat, sel


_BA = _base_addrs()
_OFFF, _OFFS = _dim_offsets()


def _sc_nn_kernel(src_hbm, inp_hbm, ba_hbm, offf_hbm, offs_hbm,
                  sel_hbm, loss_hbm,
                  src_v, inp_v, ba_v, offf_v, offs_v,
                  gc_v, t_v, i_v, sel_v, loss_v):
    nc = 2
    wid = lax.axis_index("s") * nc + lax.axis_index("c")
    b = wid // (_NQ // _QPW)
    qbase = (wid % (_NQ // _QPW)) * _QPW

    pltpu.sync_copy(src_hbm.at[b], src_v)
    pltpu.sync_copy(inp_hbm.at[b], inp_v)
    pltpu.sync_copy(ba_hbm, ba_v)
    pltpu.sync_copy(offf_hbm, offf_v)
    pltpu.sync_copy(offs_hbm, offs_v)

    lanes = lax.iota(jnp.int32, 16)
    dmask = lanes < (_D - 16)
    orows = [offs_v[k, :] for k in range(6)]

    def stage_q(qi, c):
        q = qbase + qi
        av = lax.broadcast(144 * (q // 16) + 3 * (q % 16), (16,))
        ia = av + orows[0]
        ib = av + orows[1]
        t_v[qi, pl.ds(0, 16)] = plsc.load_gather(src_v, [ia])
        t_v[qi, pl.ds(16, 16)] = plsc.load_gather(src_v, [ib])
        i_v[qi, pl.ds(0, 16)] = plsc.load_gather(inp_v, [ia])
        i_v[qi, pl.ds(16, 16)] = jnp.where(
            dmask, plsc.load_gather(inp_v, [ib]), 0.0)
        return c

    lax.fori_loop(0, _QPW, stage_q, 0)

    def stage_g(gi, c):
        base = gi * 16
        av = ba_v[pl.ds(base, 16)]
        so = ((av >= _B2).astype(jnp.int32)
              + (av >= _B4).astype(jnp.int32)) * _D
        for d in range(_D):
            offd = plsc.load_gather(offf_v, [so + d])
            gc_v[d, pl.ds(base, 16)] = plsc.load_gather(src_v, [av + offd])
        return c

    lax.fori_loop(0, _NGRP, stage_g, 0)

    def q_body(qi, lossacc):
        t_a = t_v[qi, pl.ds(0, 16)]
        t_b = t_v[qi, pl.ds(16, 16)]
        i_a = i_v[qi, pl.ds(0, 16)]
        i_b = i_v[qi, pl.ds(16, 16)]
        m_a = (t_a + i_a) * 0.5
        m_b = (t_b + i_b) * 0.5
        msp = []
        for d in range(_D):
            ms = m_a[d] if d < 16 else m_b[d - 16]
            msp.append(lax.broadcast(ms, (16,)))

        def g_body(gi, carry):
            mv, mg = carry
            accs = [jnp.zeros((16,), jnp.float32) for _ in range(4)]
            base = gi * 16
            for d in range(_D):
                gv = gc_v[d, pl.ds(base, 16)]
                df = msp[d] - gv
                accs[d % 4] = accs[d % 4] + df * df
            score = (accs[0] + accs[1]) + (accs[2] + accs[3])
            better = score < mv
            mv = jnp.where(better, score, mv)
            mg = jnp.where(better, lax.broadcast(gi, (16,)), mg)
            return mv, mg

        mv0 = jnp.full((16,), 3.0e38, jnp.float32)
        mg0 = jnp.zeros((16,), jnp.int32)
        mv, mg = lax.fori_loop(0, _NGRP, g_body, (mv0, mg0))

        gmin = jnp.min(mv)
        cand = mg * 16 + lanes
        masked = jnp.where(mv == gmin, cand, jnp.int32(2**30))
        cstar = jnp.min(masked)

        av = plsc.load_gather(ba_v, [lax.broadcast(cstar, (16,))])
        sv = (av >= _B2).astype(jnp.int32) + (av >= _B4).astype(jnp.int32)
        offa = jnp.where(sv == 0, orows[0],
                         jnp.where(sv == 1, orows[2], orows[4]))
        offb = jnp.where(sv == 0, orows[1],
                         jnp.where(sv == 1, orows[3], orows[5]))
        sela = plsc.load_gather(src_v, [av + offa])
        selb = jnp.where(dmask, plsc.load_gather(src_v, [av + offb]), 0.0)
        sel_v[qi, pl.ds(0, 16)] = sela
        sel_v[qi, pl.ds(16, 16)] = selb
        return lossacc + jnp.abs(i_a - sela) + jnp.abs(i_b - selb)

    lossacc = lax.fori_loop(0, _QPW, q_body, jnp.zeros((16,), jnp.float32))
    loss_v[...] = lossacc
    pltpu.sync_copy(sel_v, sel_hbm.at[b, pl.ds(qbase, _QPW), :])
    pltpu.sync_copy(loss_v, loss_hbm.at[wid])


def kernel(inp, tar):
    x2 = _resize(tar, 0.5)
    x4 = _resize(tar, 0.25)
    src = jnp.concatenate(
        [tar.reshape(_B, -1), x2.reshape(_B, -1), x4.reshape(_B, -1)],
        axis=1)
    src = jnp.pad(src, ((0, 0), (0, _NSRC - _NSRC0)), constant_values=1e9)
    inpf = inp.reshape(_B, -1)

    mesh = plsc.VectorSubcoreMesh(core_axis_name="c", subcore_axis_name="s")
    sel, lossp = pl.kernel(
        _sc_nn_kernel,
        mesh=mesh,
        compiler_params=pltpu.CompilerParams(
            needs_layout_passes=False, use_tc_tiling_on_sc=False),
        out_type=[
            jax.ShapeDtypeStruct((_B, _NQ, _DP), jnp.float32),
            jax.ShapeDtypeStruct((_NW, 16), jnp.float32),
        ],
        scratch_types=[
            pltpu.VMEM((_NSRC,), jnp.float32),
            pltpu.VMEM((_B2,), jnp.float32),
            pltpu.VMEM((_GP,), jnp.int32),
            pltpu.VMEM((96,), jnp.int32),
            pltpu.VMEM((6, 16), jnp.int32),
            pltpu.VMEM((_D, _GP), jnp.float32),
            pltpu.VMEM((_QPW, _DP), jnp.float32),
            pltpu.VMEM((_QPW, _DP), jnp.float32),
            pltpu.VMEM((_QPW, _DP), jnp.float32),
            pltpu.VMEM((16,), jnp.float32),
        ],
    )(src, inpf, jnp.asarray(_BA), jnp.asarray(_OFFF), jnp.asarray(_OFFS))

    selected = sel[:, :, :_D].reshape(_B, _NQ, 3, _P, _P)
    sel_img = _reassemble(selected, 48, 48)
    loss = lossp.sum() / (_B * _NQ * _D)
    return loss, sel_img


# speedup vs baseline: 4.4312x; 1.1291x over previous
import numpy as np
import jax
import jax.numpy as jnp
from jax import lax
from jax.experimental import pallas as pl
from jax.experimental.pallas import tpu as pltpu
from jax.experimental.pallas import tpu_sc as plsc

_P = 3
_D = 27
_DP = 32
_B = 4
_NQ = 256
_G = 1468
_GP = 1472
_NGRP = _GP // 16
_NW = 32
_QPW = (_B * _NQ) // _NW

_CV = 3 * 48 * 48
_NSRC = 3 * _CV


def _cubic(t):
    a = -0.75
    at = jnp.abs(t)
    w1 = ((a + 2.0) * at - (a + 3.0)) * at * at + 1.0
    w2 = (((at - 5.0) * at + 8.0) * at - 4.0) * a
    return jnp.where(at <= 1.0, w1, jnp.where(at < 2.0, w2, 0.0))


def _resize1d(x, out_size, axis):
    in_size = x.shape[axis]
    o = jnp.arange(out_size, dtype=jnp.float32)
    src = o * ((in_size - 1) / (out_size - 1))
    i0 = jnp.floor(src).astype(jnp.int32)
    ts = src - i0.astype(jnp.float32)
    offs = jnp.arange(-1, 3)
    idx = jnp.clip(i0[:, None] + offs[None, :], 0, in_size - 1)
    w = _cubic(ts[:, None] - offs[None, :].astype(jnp.float32))
    xm = jnp.moveaxis(x, axis, -1)
    g = xm[..., idx]
    res = jnp.sum(g * w, axis=-1)
    return jnp.moveaxis(res, -1, axis)


def _resize(x, scale):
    _, _, h, w = x.shape
    x = _resize1d(x, int(round(h * scale)), 2)
    x = _resize1d(x, int(round(w * scale)), 3)
    return x


def _reassemble(t, h, w, p=_P):
    b, _, c, ph, pw = t.shape
    nh, nw = h // p, w // p
    t = t.reshape(b, nh, nw, c, ph, pw)
    t = jnp.transpose(t, (0, 3, 1, 4, 2, 5))
    return t.reshape(b, c, h, w)


def _base_addrs():
    sizes = [(0, 48), (_CV, 24), (2 * _CV, 12)]
    rows = []

    def piece(base, s, i, j):
        n = (s - _P) // _P if (i or j) else s // _P
        for bi in range(n):
            for bj in range(n):
                rows.append(base + (i + bi * _P) * 48 + j + bj * _P)

    for i in range(1, _P):
        for j in range(1, _P):
            for base, s in sizes:
                piece(base, s, i, j)
    for base, s in sizes:
        piece(base, s, 0, 0)
    assert len(rows) == _G
    rows.extend([2 * _CV + 45 * 48 + 45] * (_GP - _G))
    return np.asarray(rows, np.int32)


_OFF = [c * 2304 + pi * 48 + pj
        for c in range(3) for pi in range(_P) for pj in range(_P)]
_BA = _base_addrs()
_OFFS = np.asarray(
    [_OFF[:16], _OFF[16:] + [_OFF[26]] * (_DP - _D)], np.int32)


def _sc_nn_kernel(src_hbm, inp_hbm, ba_hbm, offs_hbm,
                  sel_hbm, loss_hbm,
                  src_v, inp_v, ba_v, offs_v,
                  gc_v, t_v, i_v, sel_v, loss_v):
    nc = 2
    wid = lax.axis_index("s") * nc + lax.axis_index("c")
    b = wid // (_NQ // _QPW)
    qbase = (wid % (_NQ // _QPW)) * _QPW

    pltpu.sync_copy(src_hbm.at[b], src_v)
    pltpu.sync_copy(inp_hbm.at[b], inp_v)
    pltpu.sync_copy(ba_hbm, ba_v)
    pltpu.sync_copy(offs_hbm, offs_v)

    lanes = lax.iota(jnp.int32, 16)
    dmask = lanes < (_D - 16)
    orows = [offs_v[k, :] for k in range(2)]

    def stage_q(qi, c):
        q = qbase + qi
        av = lax.broadcast(144 * (q // 16) + 3 * (q % 16), (16,))
        ia = av + orows[0]
        ib = av + orows[1]
        t_v[qi, pl.ds(0, 16)] = plsc.load_gather(src_v, [ia])
        t_v[qi, pl.ds(16, 16)] = plsc.load_gather(src_v, [ib])
        i_v[qi, pl.ds(0, 16)] = plsc.load_gather(inp_v, [ia])
        i_v[qi, pl.ds(16, 16)] = jnp.where(
            dmask, plsc.load_gather(inp_v, [ib]), 0.0)
        return c

    lax.fori_loop(0, _QPW, stage_q, 0)

    def stage_g(gi, c):
        base = gi * 16
        av = ba_v[pl.ds(base, 16)]
        for d in range(_D):
            gc_v[d, pl.ds(base, 16)] = plsc.load_gather(src_v, [av + _OFF[d]])
        return c

    lax.fori_loop(0, _NGRP, stage_g, 0)

    def q_body(qi, lossacc):
        t_a = t_v[qi, pl.ds(0, 16)]
        t_b = t_v[qi, pl.ds(16, 16)]
        i_a = i_v[qi, pl.ds(0, 16)]
        i_b = i_v[qi, pl.ds(16, 16)]
        m_a = (t_a + i_a) * 0.5
        m_b = (t_b + i_b) * 0.5
        msp = []
        for d in range(_D):
            ms = m_a[d] if d < 16 else m_b[d - 16]
            msp.append(lax.broadcast(ms, (16,)))

        def g_body(gi, carry):
            mv, mg = carry
            accs = [jnp.zeros((16,), jnp.float32) for _ in range(4)]
            base = gi * 16
            for d in range(_D):
                gv = gc_v[d, pl.ds(base, 16)]
                df = msp[d] - gv
                accs[d % 4] = accs[d % 4] + df * df
            score = (accs[0] + accs[1]) + (accs[2] + accs[3])
            better = score < mv
            mv = jnp.where(better, score, mv)
            mg = jnp.where(better, lax.broadcast(gi, (16,)), mg)
            return mv, mg

        mv0 = jnp.full((16,), 3.0e38, jnp.float32)
        mg0 = jnp.zeros((16,), jnp.int32)
        mv, mg = lax.fori_loop(0, _NGRP, g_body, (mv0, mg0))

        gmin = jnp.min(mv)
        cand = mg * 16 + lanes
        masked = jnp.where(mv == gmin, cand, jnp.int32(2**30))
        cstar = jnp.min(masked)

        av = plsc.load_gather(ba_v, [lax.broadcast(cstar, (16,))])
        sela = plsc.load_gather(src_v, [av + orows[0]])
        selb = jnp.where(dmask, plsc.load_gather(src_v, [av + orows[1]]), 0.0)
        sel_v[qi, pl.ds(0, 16)] = sela
        sel_v[qi, pl.ds(16, 16)] = selb
        return lossacc + jnp.abs(i_a - sela) + jnp.abs(i_b - selb)

    lossacc = lax.fori_loop(0, _QPW, q_body, jnp.zeros((16,), jnp.float32))
    loss_v[...] = lossacc
    pltpu.sync_copy(sel_v, sel_hbm.at[b, pl.ds(qbase, _QPW), :])
    pltpu.sync_copy(loss_v, loss_hbm.at[wid])


def kernel(inp, tar):
    x2 = jnp.pad(_resize(tar, 0.5), ((0, 0), (0, 0), (0, 24), (0, 24)),
                 constant_values=1e9)
    x4 = jnp.pad(_resize(tar, 0.25), ((0, 0), (0, 0), (0, 36), (0, 36)),
                 constant_values=1e9)
    src = jnp.concatenate(
        [tar.reshape(_B, -1), x2.reshape(_B, -1), x4.reshape(_B, -1)],
        axis=1)
    inpf = inp.reshape(_B, -1)

    mesh = plsc.VectorSubcoreMesh(core_axis_name="c", subcore_axis_name="s")
    sel, lossp = pl.kernel(
        _sc_nn_kernel,
        mesh=mesh,
        compiler_params=pltpu.CompilerParams(
            needs_layout_passes=False, use_tc_tiling_on_sc=False),
        out_type=[
            jax.ShapeDtypeStruct((_B, _NQ, _DP), jnp.float32),
            jax.ShapeDtypeStruct((_NW, 16), jnp.float32),
        ],
        scratch_types=[
            pltpu.VMEM((_NSRC,), jnp.float32),
            pltpu.VMEM((_CV,), jnp.float32),
            pltpu.VMEM((_GP,), jnp.int32),
            pltpu.VMEM((2, 16), jnp.int32),
            pltpu.VMEM((_D, _GP), jnp.float32),
            pltpu.VMEM((_QPW, _DP), jnp.float32),
            pltpu.VMEM((_QPW, _DP), jnp.float32),
            pltpu.VMEM((_QPW, _DP), jnp.float32),
            pltpu.VMEM((16,), jnp.float32),
        ],
    )(src, inpf, jnp.asarray(_BA), jnp.asarray(_OFFS))

    selected = sel[:, :, :_D].reshape(_B, _NQ, 3, _P, _P)
    sel_img = _reassemble(selected, 48, 48)
    loss = lossp.sum() / (_B * _NQ * _D)
    return loss, sel_img
